# Initial kernel scaffold; baseline (speedup 1.0000x reference)
import jax
import jax.numpy as jnp
from jax.experimental import pallas as pl


def kernel(x, edge_attr, edge_index, params):
    raise NotImplementedError("write your pallas kernel here")


# speedup vs baseline: 2.4114x; 2.4114x over previous
import functools

import jax
import jax.numpy as jnp
from jax import lax
from jax.experimental import pallas as pl
from jax.experimental.pallas import tpu as pltpu
from jax.experimental.pallas import tpu_sc as plsc

NN = 10000
NE = 160000
D = 128
H = 8
CQ = 16
CK = 16
TEMP = 4.0

EBLK = 2000
NBLK = 2000

SC_NC = 2
SC_NS = 16
SC_NW = SC_NC * SC_NS


def _gru0_body(x_ref, wih_ref, bih_ref, bhh_ref, out_ref):
    gi = jnp.dot(x_ref[...], wih_ref[...],
                 preferred_element_type=jnp.float32) + bih_ref[...]
    gh = bhh_ref[...]
    r = jax.nn.sigmoid(gi[:, :D] + gh[:, :D])
    z = jax.nn.sigmoid(gi[:, D:2 * D] + gh[:, D:2 * D])
    n = jnp.tanh(gi[:, 2 * D:] + r * gh[:, 2 * D:])
    out_ref[...] = (1.0 - z) * n


def _gru0(x, wih_t, bih, bhh):
    nb = x.shape[0] // NBLK
    return pl.pallas_call(
        _gru0_body,
        grid=(nb,),
        in_specs=[
            pl.BlockSpec((NBLK, D), lambda i: (i, 0)),
            pl.BlockSpec((D, 3 * D), lambda i: (0, 0)),
            pl.BlockSpec((1, 3 * D), lambda i: (0, 0)),
            pl.BlockSpec((1, 3 * D), lambda i: (0, 0)),
        ],
        out_specs=pl.BlockSpec((NBLK, D), lambda i: (i, 0)),
        out_shape=jax.ShapeDtypeStruct((x.shape[0], D), jnp.float32),
    )(x, wih_t, bih, bhh)


def _edge_body(first,
               ea_ref, xi_ref, xj_ref,
               wih_ref, bih_ref, whh_ref, bhh_ref,
               w1a_ref, w1b_ref, w1c_ref, be1_ref, w2_ref, be2_ref,
               wq_ref, bq_ref, wk_ref, bk_ref, wv_ref, bv_ref,
               m1_ref, a1_ref, m2_ref, a2_ref, g_ref, gt_ref,
               enext_ref, prob_ref, value_ref):
    f32 = jnp.float32
    ea = ea_ref[...]
    xi = xi_ref[...]
    xj = xj_ref[...]

    if first:
        gi = jnp.dot(ea, wih_ref[...], preferred_element_type=f32) + bih_ref[...]
        gh = bhh_ref[...]
        r = jax.nn.sigmoid(gi[:, :D] + gh[:, :D])
        z = jax.nn.sigmoid(gi[:, D:2 * D] + gh[:, D:2 * D])
        n = jnp.tanh(gi[:, 2 * D:] + r * gh[:, 2 * D:])
        e = (1.0 - z) * n
    else:
        e = ea

    h1 = jax.nn.relu(
        jnp.dot(xi, w1a_ref[...], preferred_element_type=f32)
        + jnp.dot(e, w1b_ref[...], preferred_element_type=f32)
        + jnp.dot(xj, w1c_ref[...], preferred_element_type=f32)
        + be1_ref[...])
    emsg = jnp.dot(h1, w2_ref[...], preferred_element_type=f32) + be2_ref[...]

    q = jnp.dot(xi, wq_ref[...], preferred_element_type=f32) + bq_ref[...]
    k = jnp.dot(e, wk_ref[...], preferred_element_type=f32) + bk_ref[...]
    v = jnp.dot(xj, wv_ref[...], preferred_element_type=f32) + bv_ref[...]
    ain = jnp.concatenate([q, k], axis=1)
    hcn = jax.nn.relu(
        jnp.dot(ain, m1_ref[...], preferred_element_type=f32) + a1_ref[...])
    att = (jnp.dot(hcn, m2_ref[...], preferred_element_type=f32)
           + a2_ref[...]) * (1.0 / TEMP)

    m = jnp.max(att, axis=1, keepdims=True)
    ex = jnp.exp(att - m)
    ssum = jnp.dot(ex, g_ref[...], preferred_element_type=f32)
    denom = jnp.dot(ssum, gt_ref[...], preferred_element_type=f32)
    prob = ex / denom
    prob_ref[...] = prob
    value_ref[...] = prob * v

    msg = jax.nn.relu(emsg) if first else emsg
    gi2 = jnp.dot(msg, wih_ref[...], preferred_element_type=f32) + bih_ref[...]
    gh2 = jnp.dot(e, whh_ref[...], preferred_element_type=f32) + bhh_ref[...]
    r2 = jax.nn.sigmoid(gi2[:, :D] + gh2[:, :D])
    z2 = jax.nn.sigmoid(gi2[:, D:2 * D] + gh2[:, D:2 * D])
    n2 = jnp.tanh(gi2[:, 2 * D:] + r2 * gh2[:, 2 * D:])
    enext_ref[...] = (1.0 - z2) * n2 + z2 * e


def _edge_layer(first, ea, xi, xj, ge, lp):
    nb = NE // EBLK
    blk = lambda r, c: pl.BlockSpec((r, c), lambda i: (0, 0))
    outs = pl.pallas_call(
        functools.partial(_edge_body, first),
        grid=(nb,),
        in_specs=[
            pl.BlockSpec((EBLK, D), lambda i: (i, 0)),
            pl.BlockSpec((EBLK, D), lambda i: (i, 0)),
            pl.BlockSpec((EBLK, D), lambda i: (i, 0)),
            blk(D, 3 * D), blk(1, 3 * D), blk(D, 3 * D), blk(1, 3 * D),
            blk(D, 2 * D), blk(D, 2 * D), blk(D, 2 * D), blk(1, 2 * D),
            blk(2 * D, D), blk(1, D),
            blk(D, D), blk(1, D), blk(D, D), blk(1, D), blk(D, D), blk(1, D),
            blk(2 * D, 2 * D), blk(1, 2 * D), blk(2 * D, D), blk(1, D),
            blk(D, H), blk(H, D),
        ],
        out_specs=[
            pl.BlockSpec((EBLK, D), lambda i: (i, 0)),
            pl.BlockSpec((EBLK, D), lambda i: (i, 0)),
            pl.BlockSpec((EBLK, D), lambda i: (i, 0)),
        ],
        out_shape=[
            jax.ShapeDtypeStruct((NE, D), jnp.float32),
            jax.ShapeDtypeStruct((NE, D), jnp.float32),
            jax.ShapeDtypeStruct((NE, D), jnp.float32),
        ],
    )(ea, xi, xj,
      ge['wih'], ge['bih'], ge['whh'], ge['bhh'],
      lp['w1a'], lp['w1b'], lp['w1c'], lp['be1'], lp['w2'], lp['be2'],
      lp['wq'], lp['bq'], lp['wk'], lp['bk'], lp['wv'], lp['bv'],
      lp['m1'], lp['a1'], lp['m2'], lp['a2'], lp['g'], lp['gt'])
    return outs


def _node_body(first,
               node_ref, agga_ref, aggb_ref,
               u1a_ref, u1b_ref, u1_ref, u2_ref, u2b_ref,
               wih_ref, bih_ref, whh_ref, bhh_ref,
               out_ref):
    f32 = jnp.float32
    node = node_ref[...]
    agg = agga_ref[...] + aggb_ref[...]
    u = jax.nn.relu(
        jnp.dot(node, u1a_ref[...], preferred_element_type=f32)
        + jnp.dot(agg, u1b_ref[...], preferred_element_type=f32)
        + u1_ref[...])
    nmsg = jnp.dot(u, u2_ref[...], preferred_element_type=f32) + u2b_ref[...]
    if first:
        nmsg = jax.nn.relu(nmsg)
    gi = jnp.dot(nmsg, wih_ref[...], preferred_element_type=f32) + bih_ref[...]
    gh = jnp.dot(node, whh_ref[...], preferred_element_type=f32) + bhh_ref[...]
    r = jax.nn.sigmoid(gi[:, :D] + gh[:, :D])
    z = jax.nn.sigmoid(gi[:, D:2 * D] + gh[:, D:2 * D])
    n = jnp.tanh(gi[:, 2 * D:] + r * gh[:, 2 * D:])
    out_ref[...] = (1.0 - z) * n + z * node


def _node_layer(first, node, agg_a, agg_b, gn, lp):
    nb = NN // NBLK
    blk = lambda r, c: pl.BlockSpec((r, c), lambda i: (0, 0))
    return pl.pallas_call(
        functools.partial(_node_body, first),
        grid=(nb,),
        in_specs=[
            pl.BlockSpec((NBLK, D), lambda i: (i, 0)),
            pl.BlockSpec((NBLK, D), lambda i: (i, 0)),
            pl.BlockSpec((NBLK, D), lambda i: (i, 0)),
            blk(D, 2 * D), blk(D, 2 * D), blk(1, 2 * D),
            blk(2 * D, D), blk(1, D),
            blk(D, 3 * D), blk(1, 3 * D), blk(D, 3 * D), blk(1, 3 * D),
        ],
        out_specs=pl.BlockSpec((NBLK, D), lambda i: (i, 0)),
        out_shape=jax.ShapeDtypeStruct((NN, D), jnp.float32),
    )(node, agg_a, agg_b,
      lp['u1a'], lp['u1b'], lp['u1'], lp['u2'], lp['u2b'],
      gn['wih'], gn['bih'], gn['whh'], gn['bhh'])


G_ROWS = 2 * NE
G_PER_W = G_ROWS // SC_NW
G_CHUNK = 400
G_ITERS = G_PER_W // G_CHUNK


def _gather_body(table, idx, out, idx_v, rows_v, sem):
    c = lax.axis_index("c")
    s = lax.axis_index("s")
    wid = s * SC_NC + c
    base = wid * G_PER_W
    for i in range(G_ITERS):
        off = base + i * G_CHUNK
        pltpu.sync_copy(idx.at[pl.ds(off, G_CHUNK)], idx_v)
        pltpu.async_copy(table.at[idx_v], rows_v, sem).wait()
        pltpu.sync_copy(rows_v, out.at[pl.ds(off, G_CHUNK)])


def _sc_gather(table, idx_cat):
    call = pl.kernel(
        _gather_body,
        out_type=jax.ShapeDtypeStruct((G_ROWS, D), jnp.float32),
        mesh=plsc.VectorSubcoreMesh(core_axis_name="c", subcore_axis_name="s"),
        scratch_types=[
            pltpu.VMEM((G_CHUNK,), jnp.int32),
            pltpu.VMEM((G_CHUNK, D), jnp.float32),
            pltpu.SemaphoreType.DMA,
        ],
    )
    return call(table, idx_cat)


S_PER_SC = NE // SC_NC
S_PER_T = S_PER_SC // SC_NS
S_CHUNK = 200
S_ITERS = S_PER_T // S_CHUNK
NN_PAD = 10112
Z_ROWS = NN_PAD // SC_NS


def _scatter_body(value, idx, zrows, out, agg_sp, idx_v, val_v):
    c = lax.axis_index("c")
    s = lax.axis_index("s")
    pltpu.sync_copy(zrows, agg_sp.at[pl.ds(s * Z_ROWS, Z_ROWS)])
    plsc.subcore_barrier()
    base = c * S_PER_SC + s * S_PER_T
    for i in range(S_ITERS):
        off = base + i * S_CHUNK
        pltpu.sync_copy(idx.at[pl.ds(off, S_CHUNK)], idx_v)
        pltpu.sync_copy(value.at[pl.ds(off, S_CHUNK)], val_v)
        pltpu.sync_copy(val_v, agg_sp.at[idx_v], add=True)
    plsc.subcore_barrier()
    pltpu.sync_copy(agg_sp.at[pl.ds(s * Z_ROWS, Z_ROWS)],
                    out.at[c, pl.ds(s * Z_ROWS, Z_ROWS)])


def _sc_scatter(value, idx_i):
    call = pl.kernel(
        _scatter_body,
        out_type=jax.ShapeDtypeStruct((SC_NC, NN_PAD, D), jnp.float32),
        mesh=plsc.VectorSubcoreMesh(core_axis_name="c", subcore_axis_name="s"),
        scratch_types=[
            pltpu.VMEM_SHARED((NN_PAD, D), jnp.float32),
            pltpu.VMEM((S_CHUNK,), jnp.int32),
            pltpu.VMEM((S_CHUNK, D), jnp.float32),
        ],
    )
    zrows = jnp.zeros((Z_ROWS, D), jnp.float32)
    out = call(value, idx_i, zrows)
    return out[:, :NN]


def _prep_gru(p):
    return {
        'wih': p['Wih'].T,
        'whh': p['Whh'].T,
        'bih': p['bih'].reshape(1, -1),
        'bhh': p['bhh'].reshape(1, -1),
    }


def _prep_layer(p):
    eye = jnp.eye(H, dtype=jnp.float32)
    m1 = (p['A1'].T[:, None, :, None] * eye[None, :, None, :]).reshape(2 * D, 2 * D)
    m2 = (p['A2'].T[:, None, :, None] * eye[None, :, None, :]).reshape(2 * D, D)
    w1 = p['We1'].T
    g = jnp.tile(eye, (CK, 1))
    return {
        'w1a': w1[:D], 'w1b': w1[D:2 * D], 'w1c': w1[2 * D:],
        'be1': p['be1'].reshape(1, -1),
        'w2': p['We2'].T, 'be2': p['be2'].reshape(1, -1),
        'wq': p['Wq'].T, 'bq': p['bq'].reshape(1, -1),
        'wk': p['Wk'].T, 'bk': p['bk'].reshape(1, -1),
        'wv': p['Wv'].T, 'bv': p['bv'].reshape(1, -1),
        'm1': m1, 'a1': jnp.repeat(p['a1'], H).reshape(1, -1),
        'm2': m2, 'a2': jnp.repeat(p['a2'], H).reshape(1, -1),
        'g': g, 'gt': g.T,
        'u1a': p['U1'].T[:D], 'u1b': p['U1'].T[D:],
        'u1': p['u1'].reshape(1, -1),
        'u2': p['U2'].T, 'u2b': p['u2'].reshape(1, -1),
    }


def kernel(x, edge_attr, edge_index, params):
    idx_i = edge_index[0]
    idx_cat = jnp.concatenate([edge_index[0], edge_index[1]])
    gn = _prep_gru(params['gru_node'])
    ge = _prep_gru(params['gru_edge'])

    node = _gru0(x, gn['wih'], gn['bih'], gn['bhh'])
    edge = edge_attr
    probs = []
    for li in range(2):
        lp = _prep_layer(params['layers'][li])
        first = li == 0
        gath = _sc_gather(node, idx_cat)
        xi = gath[:NE]
        xj = gath[NE:]
        edge, prob, value = _edge_layer(first, edge, xi, xj, ge, lp)
        parts = _sc_scatter(value, idx_i)
        node = _node_layer(first, node, parts[0], parts[1], gn, lp)
        probs.append(prob.reshape(NE, CK, H))
    return node, edge, jnp.stack(probs)
